# baseline (speedup 1.0000x reference)
import functools

import jax
import jax.numpy as jnp
from jax import lax
from jax.experimental import pallas as pl
from jax.experimental.pallas import tpu as pltpu
from jax.experimental.pallas import tpu_sc as plsc

N_TIME_INTERVAL = 100
PER_TIME = 10000.0
OUTPUT_DIM = 8

B = 4096
L = 200

NUM_CORES = 2
NUM_SUBCORES = 16
NW = NUM_CORES * NUM_SUBCORES

ROWS_PER_WORKER = B // NW
ROWS_PER_GROUP = 8
GROUPS = ROWS_PER_WORKER // ROWS_PER_GROUP

TS_ROW = L + 1
OUT_ROW = L * OUTPUT_DIM
TS_GROUP = ROWS_PER_GROUP * TS_ROW
OUT_GROUP = ROWS_PER_GROUP * OUT_ROW
NVEC = 13

_mesh = plsc.VectorSubcoreMesh(core_axis_name="c", subcore_axis_name="s")


@functools.partial(
    pl.kernel,
    out_type=(
        jax.ShapeDtypeStruct((B * OUT_ROW,), jnp.float32),
        jax.ShapeDtypeStruct((B * L,), jnp.int32),
    ),
    mesh=_mesh,
    scratch_types=[
        pltpu.VMEM((TS_GROUP + 8,), jnp.int32),
        pltpu.VMEM((OUT_GROUP + 64,), jnp.float32),
        pltpu.VMEM((ROWS_PER_GROUP * L + 8,), jnp.int32),
        pltpu.VMEM((N_TIME_INTERVAL * OUTPUT_DIM,), jnp.float32),
    ],
    compiler_params=pltpu.CompilerParams(needs_layout_passes=False),
)
def _time_encode(ts_hbm, table_hbm, out_hbm, out2_hbm, ts_v, out_v, ts2_v, table_v):
    wid = lax.axis_index("s") * NUM_CORES + lax.axis_index("c")
    pltpu.sync_copy(table_hbm, table_v)

    iota = lax.iota(jnp.int32, 16)
    iota8 = iota * 8

    def per_row(r, _):
        ts_off = r * TS_ROW
        out_off = r * OUT_ROW
        ts2_off = r * L
        for v in range(NVEC):
            l0 = v * 16
            t2 = ts_v[pl.ds(ts_off + l0, 16)]
            t1 = ts_v[pl.ds(ts_off + l0 + 1, 16)]
            ts2_v[pl.ds(ts2_off + l0, 16)] = t2
            q = (t1 - t2).astype(jnp.float32) / PER_TIME
            idx = q.astype(jnp.int32)
            idx = jnp.minimum(jnp.maximum(idx, 0), N_TIME_INTERVAL - 1)
            pos = idx * OUTPUT_DIM
            epos = iota8 + (out_off + l0 * OUTPUT_DIM)
            for k in range(OUTPUT_DIM):
                vals = plsc.load_gather(table_v, [pos + k])
                plsc.store_scatter(out_v, [epos + k], vals)
        return ()

    def per_group(g, _):
        base_row = wid * ROWS_PER_WORKER + g * ROWS_PER_GROUP
        pltpu.sync_copy(
            ts_hbm.at[pl.ds(base_row * TS_ROW, TS_GROUP)],
            ts_v.at[pl.ds(0, TS_GROUP)],
        )
        lax.fori_loop(0, ROWS_PER_GROUP, per_row, (), unroll=1)
        pltpu.sync_copy(
            out_v.at[pl.ds(0, OUT_GROUP)],
            out_hbm.at[pl.ds(base_row * OUT_ROW, OUT_GROUP)],
        )
        pltpu.sync_copy(
            ts2_v.at[pl.ds(0, ROWS_PER_GROUP * L)],
            out2_hbm.at[pl.ds(base_row * L, ROWS_PER_GROUP * L)],
        )
        return ()

    lax.fori_loop(0, GROUPS, per_group, (), unroll=1)


def kernel(input, timestamp, train, W, b):
    del input, train
    table = (W.T + b[None, :]).astype(jnp.float32).reshape(-1)
    ts_flat = timestamp.astype(jnp.int32).reshape(-1)
    out, out2 = _time_encode(ts_flat, table)
    return (out.reshape(B, L, OUTPUT_DIM), out2.reshape(B, L).astype(timestamp.dtype))


# speedup vs baseline: 4.9463x; 4.9463x over previous
import functools

import jax
import jax.numpy as jnp
from jax import lax
from jax.experimental import pallas as pl
from jax.experimental.pallas import tpu as pltpu
from jax.experimental.pallas import tpu_sc as plsc

N_TIME_INTERVAL = 100
PER_TIME = 10000.0
OUTPUT_DIM = 8

B = 4096
L = 200
TS_ROW = L + 1

NUM_CORES = 2
NUM_SUBCORES = 16
NW = NUM_CORES * NUM_SUBCORES
BPW = B // NW

L_CHUNK = 40
N_CHUNKS = L // L_CHUNK

_mesh = plsc.VectorSubcoreMesh(core_axis_name="c", subcore_axis_name="s")


@functools.partial(
    pl.kernel,
    out_type=(
        jax.ShapeDtypeStruct((L, NW, OUTPUT_DIM, BPW), jnp.float32),
        jax.ShapeDtypeStruct((L // 8, NW, 8, BPW), jnp.int32),
    ),
    mesh=_mesh,
    scratch_types=[
        pltpu.VMEM((BPW * TS_ROW,), jnp.int32),
        pltpu.VMEM((L_CHUNK, OUTPUT_DIM, BPW), jnp.float32),
        pltpu.VMEM((L_CHUNK // 8, 8, BPW), jnp.int32),
        pltpu.VMEM((N_TIME_INTERVAL * OUTPUT_DIM,), jnp.float32),
    ],
    compiler_params=pltpu.CompilerParams(needs_layout_passes=False),
)
def _time_encode(ts_hbm, table_hbm, out_hbm, out2_hbm, ts_v, out_v, out2_v, table_v):
    wid = lax.axis_index("s") * NUM_CORES + lax.axis_index("c")
    pltpu.sync_copy(table_hbm, table_v)
    pltpu.sync_copy(ts_hbm.at[pl.ds(wid * (BPW * TS_ROW), BPW * TS_ROW)], ts_v)

    iota = lax.iota(jnp.int32, 16)

    def chunk(c, _):
        l0 = c * L_CHUNK
        for v in range(BPW // 16):
            base = (iota + v * 16) * TS_ROW

            def step(lr, t_prev):
                t_cur = plsc.load_gather(ts_v, [base + (l0 + lr + 1)])
                q = (t_cur - t_prev).astype(jnp.float32) / PER_TIME
                idx = q.astype(jnp.int32)
                idx = jnp.minimum(jnp.maximum(idx, 0), N_TIME_INTERVAL - 1)
                pos = idx * OUTPUT_DIM
                for k in range(OUTPUT_DIM):
                    out_v[lr, k, pl.ds(v * 16, 16)] = plsc.load_gather(
                        table_v, [pos + k])
                out2_v[lr // 8, lr % 8, pl.ds(v * 16, 16)] = t_prev
                return t_cur

            t0 = plsc.load_gather(ts_v, [base + l0])
            lax.fori_loop(0, L_CHUNK, step, t0, unroll=2)

        pltpu.sync_copy(out_v, out_hbm.at[pl.ds(l0, L_CHUNK), wid, :, :])
        pltpu.sync_copy(out2_v, out2_hbm.at[pl.ds(c * (L_CHUNK // 8), L_CHUNK // 8), wid, :, :])
        return ()

    lax.fori_loop(0, N_CHUNKS, chunk, (), unroll=1)


def kernel(input, timestamp, train, W, b):
    del input, train
    table = (W.T + b[None, :]).astype(jnp.float32).reshape(-1)
    ts_flat = timestamp.astype(jnp.int32).reshape(-1)
    buf, buf2 = _time_encode(ts_flat, table)
    out = buf.transpose(1, 3, 0, 2).reshape(B, L, OUTPUT_DIM)
    out2 = buf2.transpose(1, 3, 0, 2).reshape(B, L).astype(timestamp.dtype)
    return (out, out2)


# speedup vs baseline: 5.0316x; 1.0172x over previous
import functools

import jax
import jax.numpy as jnp
from jax import lax
from jax.experimental import pallas as pl
from jax.experimental.pallas import tpu as pltpu
from jax.experimental.pallas import tpu_sc as plsc

N_TIME_INTERVAL = 100
PER_TIME = 10000.0
OUTPUT_DIM = 8

B = 4096
L = 200
TS_ROW = L + 1

NUM_CORES = 2
NUM_SUBCORES = 16
NW = NUM_CORES * NUM_SUBCORES
BPW = B // NW

L_CHUNK = 40
N_CHUNKS = L // L_CHUNK

_mesh = plsc.VectorSubcoreMesh(core_axis_name="c", subcore_axis_name="s")


@functools.partial(
    pl.kernel,
    out_type=(
        jax.ShapeDtypeStruct((L, NW, OUTPUT_DIM, BPW), jnp.float32),
        jax.ShapeDtypeStruct((L // 8, NW, 8, BPW), jnp.int32),
    ),
    mesh=_mesh,
    scratch_types=[
        pltpu.VMEM((BPW * TS_ROW,), jnp.int32),
        pltpu.VMEM((L_CHUNK, OUTPUT_DIM, BPW), jnp.float32),
        pltpu.VMEM((L_CHUNK // 8, 8, BPW), jnp.int32),
        pltpu.VMEM((N_TIME_INTERVAL * OUTPUT_DIM,), jnp.float32),
    ],
    compiler_params=pltpu.CompilerParams(needs_layout_passes=False),
)
def _time_encode(ts_hbm, table_hbm, out_hbm, out2_hbm, ts_v, out_v, out2_v, table_v):
    wid = lax.axis_index("s") * NUM_CORES + lax.axis_index("c")
    pltpu.sync_copy(table_hbm, table_v)
    pltpu.sync_copy(ts_hbm.at[pl.ds(wid * (BPW * TS_ROW), BPW * TS_ROW)], ts_v)

    iota = lax.iota(jnp.int32, 16)

    iota201 = iota * TS_ROW

    def chunk(c, _):
        l0 = c * L_CHUNK

        def lane_group(v, _):
            v16 = v * 16
            base = iota201 + (v16 * TS_ROW + l0)
            t_prev = plsc.load_gather(ts_v, [base])
            for lr in range(L_CHUNK):
                t_cur = plsc.load_gather(ts_v, [base + (lr + 1)])
                q = (t_cur - t_prev).astype(jnp.float32) / PER_TIME
                idx = q.astype(jnp.int32)
                idx = jnp.minimum(jnp.maximum(idx, 0), N_TIME_INTERVAL - 1)
                pos = idx * OUTPUT_DIM
                for k in range(OUTPUT_DIM):
                    out_v[lr, k, pl.ds(v16, 16)] = plsc.load_gather(
                        table_v, [pos + k])
                out2_v[lr // 8, lr % 8, pl.ds(v16, 16)] = t_prev
                t_prev = t_cur
            return ()

        lax.fori_loop(0, BPW // 16, lane_group, (), unroll=1)

        pltpu.sync_copy(out_v, out_hbm.at[pl.ds(l0, L_CHUNK), wid, :, :])
        pltpu.sync_copy(out2_v, out2_hbm.at[pl.ds(c * (L_CHUNK // 8), L_CHUNK // 8), wid, :, :])
        return ()

    lax.fori_loop(0, N_CHUNKS, chunk, (), unroll=1)


def kernel(input, timestamp, train, W, b):
    del input, train
    table = (W.T + b[None, :]).astype(jnp.float32).reshape(-1)
    ts_flat = timestamp.astype(jnp.int32).reshape(-1)
    buf, buf2 = _time_encode(ts_flat, table)
    out = buf.transpose(1, 3, 0, 2).reshape(B, L, OUTPUT_DIM)
    out2 = buf2.transpose(1, 3, 0, 2).reshape(B, L).astype(timestamp.dtype)
    return (out, out2)


# speedup vs baseline: 5.5100x; 1.0951x over previous
import functools

import jax
import jax.numpy as jnp
from jax import lax
from jax.experimental import pallas as pl
from jax.experimental.pallas import tpu as pltpu
from jax.experimental.pallas import tpu_sc as plsc

N_TIME_INTERVAL = 100
PER_TIME = 10000.0
OUTPUT_DIM = 8

B = 4096
L = 200
TS_ROW = L + 1

NUM_CORES = 2
NUM_SUBCORES = 16
NW = NUM_CORES * NUM_SUBCORES
BPW = B // NW

L_CHUNK = 40
N_CHUNKS = L // L_CHUNK

_mesh = plsc.VectorSubcoreMesh(core_axis_name="c", subcore_axis_name="s")


@functools.partial(
    pl.kernel,
    out_type=(
        jax.ShapeDtypeStruct((L, NW, OUTPUT_DIM, BPW), jnp.float32),
        jax.ShapeDtypeStruct((L // 8, NW, 8, BPW), jnp.int32),
    ),
    mesh=_mesh,
    scratch_types=[
        pltpu.VMEM((BPW * TS_ROW,), jnp.int32),
        pltpu.VMEM((L_CHUNK, OUTPUT_DIM, BPW), jnp.float32),
        pltpu.VMEM((L_CHUNK // 8, 8, BPW), jnp.int32),
        pltpu.VMEM((N_TIME_INTERVAL * OUTPUT_DIM,), jnp.float32),
    ],
    compiler_params=pltpu.CompilerParams(needs_layout_passes=False),
)
def _time_encode(ts_hbm, table_hbm, out_hbm, out2_hbm, ts_v, out_v, out2_v, table_v):
    wid = lax.axis_index("s") * NUM_CORES + lax.axis_index("c")
    pltpu.sync_copy(table_hbm, table_v)
    pltpu.sync_copy(ts_hbm.at[pl.ds(wid * (BPW * TS_ROW), BPW * TS_ROW)], ts_v)

    iota = lax.iota(jnp.int32, 16)

    iota201 = iota * TS_ROW
    NI = 2

    def chunk(c, _):
        l0 = c * L_CHUNK

        def lane_pair(p, _):
            v16 = [(p * NI + g) * 16 for g in range(NI)]
            base = [iota201 + (v16[g] * TS_ROW + l0) for g in range(NI)]
            t_prev = [plsc.load_gather(ts_v, [base[g]]) for g in range(NI)]
            for lr in range(L_CHUNK):
                t_cur = [plsc.load_gather(ts_v, [base[g] + (lr + 1)])
                         for g in range(NI)]
                q = [(t_cur[g] - t_prev[g]).astype(jnp.float32) / PER_TIME
                     for g in range(NI)]
                idx = [q[g].astype(jnp.int32) for g in range(NI)]
                idx = [jnp.minimum(jnp.maximum(idx[g], 0), N_TIME_INTERVAL - 1)
                       for g in range(NI)]
                pos = [idx[g] * OUTPUT_DIM for g in range(NI)]
                vals = [[plsc.load_gather(table_v, [pos[g] + k])
                         for g in range(NI)] for k in range(OUTPUT_DIM)]
                for k in range(OUTPUT_DIM):
                    for g in range(NI):
                        out_v[lr, k, pl.ds(v16[g], 16)] = vals[k][g]
                for g in range(NI):
                    out2_v[lr // 8, lr % 8, pl.ds(v16[g], 16)] = t_prev[g]
                    t_prev[g] = t_cur[g]
            return ()

        lax.fori_loop(0, BPW // 16 // NI, lane_pair, (), unroll=1)

        pltpu.sync_copy(out_v, out_hbm.at[pl.ds(l0, L_CHUNK), wid, :, :])
        pltpu.sync_copy(out2_v, out2_hbm.at[pl.ds(c * (L_CHUNK // 8), L_CHUNK // 8), wid, :, :])
        return ()

    lax.fori_loop(0, N_CHUNKS, chunk, (), unroll=1)


def kernel(input, timestamp, train, W, b):
    del input, train
    table = (W.T + b[None, :]).astype(jnp.float32).reshape(-1)
    ts_flat = timestamp.astype(jnp.int32).reshape(-1)
    buf, buf2 = _time_encode(ts_flat, table)
    out = buf.transpose(1, 3, 0, 2).reshape(B, L, OUTPUT_DIM)
    out2 = buf2.transpose(1, 3, 0, 2).reshape(B, L).astype(timestamp.dtype)
    return (out, out2)


# speedup vs baseline: 10.4263x; 1.8923x over previous
import functools

import jax
import jax.numpy as jnp
from jax import lax
from jax.experimental import pallas as pl
from jax.experimental.pallas import tpu as pltpu
from jax.experimental.pallas import tpu_sc as plsc

N_TIME_INTERVAL = 100
PER_TIME = 10000.0
OUTPUT_DIM = 8

B = 4096
L = 200
TS_ROW = L + 1

NUM_CORES = 2
NUM_SUBCORES = 16
NW = NUM_CORES * NUM_SUBCORES
BPW = B // NW

L_CHUNK = 8
N_CHUNKS = L // L_CHUNK
NI = 4

_mesh = plsc.VectorSubcoreMesh(core_axis_name="c", subcore_axis_name="s")


@functools.partial(
    pl.kernel,
    out_type=(
        jax.ShapeDtypeStruct((L, NW, OUTPUT_DIM, BPW), jnp.float32),
        jax.ShapeDtypeStruct((L // 8, NW, 8, BPW), jnp.int32),
    ),
    mesh=_mesh,
    scratch_types=[
        pltpu.VMEM((BPW * TS_ROW,), jnp.int32),
        pltpu.VMEM((2, L_CHUNK, OUTPUT_DIM, BPW), jnp.float32),
        pltpu.VMEM((2, L_CHUNK // 8, 8, BPW), jnp.int32),
        pltpu.VMEM((N_TIME_INTERVAL * OUTPUT_DIM,), jnp.float32),
        pltpu.SemaphoreType.DMA((2,)),
    ],
    compiler_params=pltpu.CompilerParams(needs_layout_passes=False),
)
def _time_encode(ts_hbm, table_hbm, out_hbm, out2_hbm,
                 ts_v, out_v, out2_v, table_v, sem):
    wid = lax.axis_index("s") * NUM_CORES + lax.axis_index("c")
    pltpu.sync_copy(table_hbm, table_v)
    pltpu.sync_copy(ts_hbm.at[pl.ds(wid * (BPW * TS_ROW), BPW * TS_ROW)], ts_v)

    iota = lax.iota(jnp.int32, 16)
    iota201 = iota * TS_ROW
    NG = BPW // 16

    def chunk(c, buf):
        l0 = c * L_CHUNK

        @pl.when(c >= 2)
        def _drain():
            pltpu.make_async_copy(
                out_v.at[buf], out_hbm.at[pl.ds(0, L_CHUNK), 0, :, :],
                sem.at[buf]).wait()
            pltpu.make_async_copy(
                out2_v.at[buf], out2_hbm.at[pl.ds(0, L_CHUNK // 8), 0, :, :],
                sem.at[buf]).wait()

        def lane_pack(p, _):
            v16 = [(p * NI + g) * 16 for g in range(NI)]
            base = [iota201 + (v16[g] * TS_ROW + l0) for g in range(NI)]
            t_prev = [plsc.load_gather(ts_v, [base[g]]) for g in range(NI)]
            for lr in range(L_CHUNK):
                t_cur = [plsc.load_gather(ts_v, [base[g] + (lr + 1)])
                         for g in range(NI)]
                q = [(t_cur[g] - t_prev[g]).astype(jnp.float32) / PER_TIME
                     for g in range(NI)]
                idx = [q[g].astype(jnp.int32) for g in range(NI)]
                idx = [jnp.minimum(jnp.maximum(idx[g], 0), N_TIME_INTERVAL - 1)
                       for g in range(NI)]
                pos = [idx[g] * OUTPUT_DIM for g in range(NI)]
                prev = None
                for k in range(OUTPUT_DIM):
                    cur = [plsc.load_gather(table_v, [pos[g] + k])
                           for g in range(NI)]
                    if prev is not None:
                        for g in range(NI):
                            out_v[buf, lr, k - 1, pl.ds(v16[g], 16)] = prev[g]
                    prev = cur
                for g in range(NI):
                    out_v[buf, lr, OUTPUT_DIM - 1, pl.ds(v16[g], 16)] = prev[g]
                for g in range(NI):
                    out2_v[buf, lr // 8, lr % 8, pl.ds(v16[g], 16)] = t_prev[g]
                    t_prev[g] = t_cur[g]
            return ()

        lax.fori_loop(0, NG // NI, lane_pack, (), unroll=1)

        pltpu.async_copy(
            out_v.at[buf], out_hbm.at[pl.ds(l0, L_CHUNK), wid, :, :],
            sem.at[buf])
        pltpu.async_copy(
            out2_v.at[buf],
            out2_hbm.at[pl.ds(c * (L_CHUNK // 8), L_CHUNK // 8), wid, :, :],
            sem.at[buf])

    def chunk_pair(i, _):
        chunk(2 * i, 0)
        chunk(2 * i + 1, 1)
        return ()

    lax.fori_loop(0, (N_CHUNKS - 1) // 2, chunk_pair, (), unroll=1)
    chunk(jnp.int32(N_CHUNKS - 1), 0)

    for buf in range(2):
        pltpu.make_async_copy(
            out_v.at[buf], out_hbm.at[pl.ds(0, L_CHUNK), 0, :, :],
            sem.at[buf]).wait()
        pltpu.make_async_copy(
            out2_v.at[buf], out2_hbm.at[pl.ds(0, L_CHUNK // 8), 0, :, :],
            sem.at[buf]).wait()


def kernel(input, timestamp, train, W, b):
    del input, train
    table = (W.T + b[None, :]).astype(jnp.float32).reshape(-1)
    ts_flat = timestamp.astype(jnp.int32).reshape(-1)
    buf, buf2 = _time_encode(ts_flat, table)
    out = buf.transpose(1, 3, 0, 2).reshape(B, L, OUTPUT_DIM)
    out2 = buf2.transpose(1, 3, 0, 2).reshape(B, L).astype(timestamp.dtype)
    return (out, out2)


# speedup vs baseline: 13.0982x; 1.2563x over previous
import functools

import jax
import jax.numpy as jnp
from jax import lax
from jax.experimental import pallas as pl
from jax.experimental.pallas import tpu as pltpu
from jax.experimental.pallas import tpu_sc as plsc

N_TIME_INTERVAL = 100
PER_TIME = 10000.0
OUTPUT_DIM = 8

B = 4096
L = 200
TS_ROW = L + 1

NUM_CORES = 2
NUM_SUBCORES = 16
NW = NUM_CORES * NUM_SUBCORES
BPW = B // NW

L_CHUNK = 8
N_CHUNKS = L // L_CHUNK
NI = 4

_mesh = plsc.VectorSubcoreMesh(core_axis_name="c", subcore_axis_name="s")


@functools.partial(
    pl.kernel,
    out_type=(
        jax.ShapeDtypeStruct((L, NW, OUTPUT_DIM, BPW), jnp.float32),
        jax.ShapeDtypeStruct((L, B), jnp.int32),
    ),
    mesh=_mesh,
    scratch_types=[
        pltpu.VMEM((L, BPW), jnp.int32),
        pltpu.VMEM((BPW,), jnp.int32),
        pltpu.VMEM((2, L_CHUNK, OUTPUT_DIM, BPW), jnp.float32),
        pltpu.VMEM((N_TIME_INTERVAL * OUTPUT_DIM,), jnp.float32),
        pltpu.SemaphoreType.DMA((2,)),
        pltpu.SemaphoreType.DMA,
    ],
    compiler_params=pltpu.CompilerParams(
        needs_layout_passes=False, use_tc_tiling_on_sc=True),
)
def _time_encode(ts_hbm, ts_last_hbm, table_hbm, out_hbm, out2_hbm,
                 ts_v, ts_last_v, out_v, table_v, sem, sem2):
    wid = lax.axis_index("s") * NUM_CORES + lax.axis_index("c")
    b0 = wid * BPW
    pltpu.sync_copy(table_hbm, table_v)
    pltpu.sync_copy(ts_hbm.at[pl.ds(0, L), pl.ds(b0, BPW)], ts_v)
    pltpu.sync_copy(ts_last_hbm.at[pl.ds(b0, BPW)], ts_last_v)
    out2_cp = pltpu.async_copy(ts_v, out2_hbm.at[:, pl.ds(b0, BPW)], sem2)

    NG = BPW // 16

    def chunk(c, buf, last=False):
        l0 = c * L_CHUNK

        @pl.when(c >= 2)
        def _drain():
            pltpu.make_async_copy(
                out_v.at[buf], out_hbm.at[pl.ds(0, L_CHUNK), 0, :, :],
                sem.at[buf]).wait()

        def lane_pack(p, _):
            v16 = [(p * NI + g) * 16 for g in range(NI)]
            t_prev = [ts_v[l0, pl.ds(v16[g], 16)] for g in range(NI)]
            for lr in range(L_CHUNK):
                if last and lr == L_CHUNK - 1:
                    t_cur = [ts_last_v[pl.ds(v16[g], 16)] for g in range(NI)]
                else:
                    t_cur = [ts_v[l0 + lr + 1, pl.ds(v16[g], 16)]
                             for g in range(NI)]
                q = [(t_cur[g] - t_prev[g]).astype(jnp.float32) / PER_TIME
                     for g in range(NI)]
                idx = [q[g].astype(jnp.int32) for g in range(NI)]
                idx = [jnp.minimum(jnp.maximum(idx[g], 0), N_TIME_INTERVAL - 1)
                       for g in range(NI)]
                pos = [idx[g] * OUTPUT_DIM for g in range(NI)]
                prev = None
                for k in range(OUTPUT_DIM):
                    cur = []
                    for g in range(NI):
                        cur.append(plsc.load_gather(table_v, [pos[g] + k]))
                        if prev is not None:
                            out_v[buf, lr, k - 1, pl.ds(v16[g], 16)] = prev[g]
                    prev = cur
                for g in range(NI):
                    out_v[buf, lr, OUTPUT_DIM - 1, pl.ds(v16[g], 16)] = prev[g]
                    t_prev[g] = t_cur[g]
            return ()

        lax.fori_loop(0, NG // NI, lane_pack, (), unroll=1)

        pltpu.async_copy(
            out_v.at[buf], out_hbm.at[pl.ds(l0, L_CHUNK), wid, :, :],
            sem.at[buf])

    def chunk_pair(i, _):
        chunk(2 * i, 0)
        chunk(2 * i + 1, 1)
        return ()

    lax.fori_loop(0, (N_CHUNKS - 1) // 2, chunk_pair, (), unroll=1)
    chunk(N_CHUNKS - 1, 0, last=True)

    for buf in range(2):
        pltpu.make_async_copy(
            out_v.at[buf], out_hbm.at[pl.ds(0, L_CHUNK), 0, :, :],
            sem.at[buf]).wait()
    out2_cp.wait()


def kernel(input, timestamp, train, W, b):
    del input, train
    table = (W.T + b[None, :]).astype(jnp.float32).reshape(-1)
    tsi = timestamp.astype(jnp.int32)
    tst = tsi.T
    buf, buf2 = _time_encode(tst, tsi[:, L], table)
    out = buf.transpose(1, 3, 0, 2).reshape(B, L, OUTPUT_DIM)
    out2 = buf2.T.astype(timestamp.dtype)
    return (out, out2)
